# baseline (speedup 1.0000x reference)
import jax
import jax.numpy as jnp
from jax import lax
from jax.experimental import pallas as pl
from jax.experimental.pallas import tpu as pltpu
from jax.experimental.pallas import tpu_sc as plsc

NC = 2
NS = 16
CHUNK = 16
UNROLL = 8


def _body(seq_hbm, tab_hbm, out_hbm,
          row_v, idx_v, ob_v,
          stage_sp,
          sem_row, sem_out0, sem_out1):
    c = lax.axis_index("c")
    s = lax.axis_index("s")
    B = idx_v.shape[0]
    L = seq_hbm.shape[0] // B
    V = row_v.shape[0]
    D = tab_hbm.shape[0] // V
    HB = B // 2
    d_per_sc = D // NC
    d_per_phase = NS
    phases = d_per_sc // d_per_phase
    out_sems = (sem_out0, sem_out1)

    def gather_half(h):
        def blk(ci, carry):
            base = ci * (CHUNK * UNROLL)
            for u in range(UNROLL):
                off = pl.multiple_of(base + u * CHUNK, CHUNK)
                iv = idx_v[pl.ds(h * HB + off, CHUNK)]
                ob_v[pl.ds(off, CHUNK)] = plsc.load_gather(row_v, [iv])
            return carry

        lax.fori_loop(0, HB // (CHUNK * UNROLL), blk, 0)

    def out_slice(l, dbase, h):
        return out_hbm.at[l, pl.ds(dbase, d_per_phase), pl.ds(h * HB, HB)]

    for p in range(phases):
        dbase = c * d_per_sc + p * d_per_phase

        pltpu.async_copy(
            tab_hbm.at[pl.ds((dbase + s) * V, V)], row_v, sem_row
        ).wait()

        def do_l(l, wait_prev, dbase=dbase):
            pltpu.sync_copy(seq_hbm.at[pl.ds(l * B, B)], idx_v)
            for h in range(2):
                if wait_prev:
                    @pl.when(s == 0)
                    def _(h=h, l=l):
                        pltpu.make_async_copy(
                            stage_sp.at[h], out_slice(l - 1, dbase, h),
                            out_sems[h],
                        ).wait()
                plsc.subcore_barrier()
                gather_half(h)
                pltpu.sync_copy(ob_v, stage_sp.at[h, s])
                plsc.subcore_barrier()

                @pl.when(s == 0)
                def _(h=h, l=l):
                    pltpu.async_copy(
                        stage_sp.at[h], out_slice(l, dbase, h), out_sems[h]
                    )

        do_l(0, False)
        lax.fori_loop(1, L, lambda l, cr, f=do_l: (f(l, True), cr)[1], 0)

        @pl.when(s == 0)
        def _(dbase=dbase):
            for h in range(2):
                pltpu.make_async_copy(
                    stage_sp.at[h], out_slice(L - 1, dbase, h), out_sems[h]
                ).wait()
        plsc.subcore_barrier()


def kernel(seq, hidden, table):
    B, L = seq.shape
    V, D = table.shape
    seq_flat = seq.T.reshape(L * B)
    tab_flat = table.T.reshape(D * V)

    mesh = plsc.VectorSubcoreMesh(core_axis_name="c", subcore_axis_name="s")
    out_t = pl.kernel(
        _body,
        out_type=jax.ShapeDtypeStruct((L, D, B), jnp.float32),
        mesh=mesh,
        scratch_types=[
            pltpu.VMEM((V,), jnp.float32),
            pltpu.VMEM((B,), jnp.int32),
            pltpu.VMEM((B // 2,), jnp.float32),
            pltpu.VMEM_SHARED((2, NS, B // 2), jnp.float32),
            pltpu.SemaphoreType.DMA,
            pltpu.SemaphoreType.DMA,
            pltpu.SemaphoreType.DMA,
        ],
        compiler_params=pltpu.CompilerParams(needs_layout_passes=False),
    )(seq_flat, tab_flat)
    return out_t.transpose(2, 0, 1)


# speedup vs baseline: 1.2259x; 1.2259x over previous
import jax
import jax.numpy as jnp
from jax import lax
from jax.experimental import pallas as pl
from jax.experimental.pallas import tpu as pltpu
from jax.experimental.pallas import tpu_sc as plsc

NC = 2
NS = 16
CHUNK = 16
UNROLL = 8


def _body(seq_hbm, tab_hbm, out_hbm,
          row_v, idx_v, ob_v,
          stage_sp,
          sem_row, sem_out0, sem_out1):
    c = lax.axis_index("c")
    s = lax.axis_index("s")
    B = idx_v.shape[0]
    L = seq_hbm.shape[0] // B
    V = row_v.shape[0]
    D = tab_hbm.shape[0] // V
    HB = B // 2
    d_per_sc = D // NC
    d_per_phase = NS
    phases = d_per_sc // d_per_phase
    out_sems = (sem_out0, sem_out1)

    def gather_full():
        @plsc.parallel_loop(0, B // CHUNK, unroll=UNROLL)
        def _(ci):
            off = pl.multiple_of(ci * CHUNK, CHUNK)
            iv = idx_v[pl.ds(off, CHUNK)]
            ob_v[pl.ds(off, CHUNK)] = plsc.load_gather(row_v, [iv])

    def out_slice(l, dbase):
        return out_hbm.at[l, pl.ds(dbase, d_per_phase), :]

    for p in range(phases):
        dbase = c * d_per_sc + p * d_per_phase

        pltpu.async_copy(
            tab_hbm.at[pl.ds((dbase + s) * V, V)], row_v, sem_row
        ).wait()

        def do_pair(j, wait_prev, dbase=dbase):
            for k in range(2):
                l = 2 * j + k
                pltpu.sync_copy(seq_hbm.at[pl.ds(l * B, B)], idx_v)
                gather_full()
                if wait_prev:
                    @pl.when(s == 0)
                    def _(k=k, l=l):
                        pltpu.make_async_copy(
                            stage_sp.at[k], out_slice(l - 2, dbase),
                            out_sems[k],
                        ).wait()
                plsc.subcore_barrier()
                pltpu.sync_copy(ob_v, stage_sp.at[k, s])
                plsc.subcore_barrier()

                @pl.when(s == 0)
                def _(k=k, l=l):
                    pltpu.async_copy(
                        stage_sp.at[k], out_slice(l, dbase), out_sems[k]
                    )

        do_pair(0, False)
        lax.fori_loop(1, L // 2, lambda j, cr, f=do_pair: (f(j, True), cr)[1],
                      0)

        @pl.when(s == 0)
        def _(dbase=dbase):
            for k in range(2):
                pltpu.make_async_copy(
                    stage_sp.at[k], out_slice(L - 2 + k, dbase), out_sems[k]
                ).wait()
        plsc.subcore_barrier()


def kernel(seq, hidden, table):
    B, L = seq.shape
    V, D = table.shape
    seq_flat = seq.T.reshape(L * B)
    tab_flat = table.T.reshape(D * V)

    mesh = plsc.VectorSubcoreMesh(core_axis_name="c", subcore_axis_name="s")
    out_t = pl.kernel(
        _body,
        out_type=jax.ShapeDtypeStruct((L, D, B), jnp.float32),
        mesh=mesh,
        scratch_types=[
            pltpu.VMEM((V,), jnp.float32),
            pltpu.VMEM((B,), jnp.int32),
            pltpu.VMEM((B,), jnp.float32),
            pltpu.VMEM_SHARED((2, NS, B), jnp.float32),
            pltpu.SemaphoreType.DMA,
            pltpu.SemaphoreType.DMA,
            pltpu.SemaphoreType.DMA,
        ],
        compiler_params=pltpu.CompilerParams(needs_layout_passes=False),
    )(seq_flat, tab_flat)
    return out_t.transpose(2, 0, 1)


# speedup vs baseline: 1.5562x; 1.2694x over previous
import jax
import jax.numpy as jnp
from jax import lax
from jax.experimental import pallas as pl
from jax.experimental.pallas import tpu as pltpu
from jax.experimental.pallas import tpu_sc as plsc

NUM_CORES = 2
NUM_SUBCORES = 16
NUM_WORKERS = NUM_CORES * NUM_SUBCORES

GROUP = 128


GPB = 5
NBUF = 2


def _gather_body(idx_hbm, table_hbm, out_hbm, idx_v,
                 rows0, rows1, gsem0, gsem1, wsem0, wsem1):
    c = lax.axis_index("c")
    s = lax.axis_index("s")
    wid = s * NUM_CORES + c
    n_per_w = idx_hbm.shape[0] // NUM_WORKERS
    groups_per_w = n_per_w // GROUP
    n_blocks = groups_per_w // GPB
    block_rows = GPB * GROUP
    base = wid * n_per_w
    pltpu.sync_copy(idx_hbm.at[pl.ds(base, n_per_w)], idx_v)

    bufs = (rows0, rows1)
    gsems = (gsem0, gsem1)
    wsems = (wsem0, wsem1)
    wb = [None, None]

    for b in range(n_blocks):
        k = b % NBUF
        if wb[k] is not None:
            wb[k].wait()
        gd = []
        for g in range(GPB):
            jg = b * GPB + g
            idx_g = idx_v.at[pl.ds(jg * GROUP, GROUP)]
            dst = bufs[k].at[pl.ds(g * GROUP, GROUP)]
            gd.append(pltpu.async_copy(table_hbm.at[idx_g], dst, gsems[k]))
        for d in gd:
            d.wait()
        wb[k] = pltpu.async_copy(
            bufs[k], out_hbm.at[pl.ds(base + b * block_rows, block_rows)],
            wsems[k])
    for k in range(NBUF):
        if wb[k] is not None:
            wb[k].wait()


def kernel(seq, hidden, table):
    B, L = seq.shape
    V, D = table.shape
    N = B * L
    idx = seq.reshape(N)
    n_per_w = N // NUM_WORKERS

    mesh = plsc.VectorSubcoreMesh(core_axis_name="c", subcore_axis_name="s")
    out = pl.kernel(
        _gather_body,
        out_type=jax.ShapeDtypeStruct((N, D), jnp.float32),
        mesh=mesh,
        scratch_types=[
            pltpu.VMEM((n_per_w,), jnp.int32),
            pltpu.VMEM((GPB * GROUP, D), jnp.float32),
            pltpu.VMEM((GPB * GROUP, D), jnp.float32),
            pltpu.SemaphoreType.DMA,
            pltpu.SemaphoreType.DMA,
            pltpu.SemaphoreType.DMA,
            pltpu.SemaphoreType.DMA,
        ],
        compiler_params=pltpu.CompilerParams(use_tc_tiling_on_sc=False),
    )(idx, table)
    return out.reshape(B, L, D)
